# Initial kernel scaffold; baseline (speedup 1.0000x reference)
import jax
import jax.numpy as jnp
from jax.experimental import pallas as pl


def kernel(x_SB, x_PV, x_PQ, x_NB, ei_SB_PV, ei_SB_PQ, ei_SB_NB, ei_PV_PQ, ei_PV_NB, ei_PQ_NB, ei_PV_SB, ei_PQ_SB, ei_NB_SB, ei_PQ_PV, ei_NB_PQ, ei_PV_PV, ei_PQ_PQ, ei_NB_NB, ea_SB_PV, ea_SB_PQ, ea_SB_NB, ea_PV_PQ, ea_PV_NB, ea_PQ_NB, ea_PV_SB, ea_PQ_SB, ea_NB_SB, ea_PQ_PV, ea_NB_PQ, ea_PV_PV, ea_PQ_PQ, ea_NB_NB, Wq, Wk, Wv, Ws, We, bq, bk, bv, bs):
    raise NotImplementedError("write your pallas kernel here")


# speedup vs baseline: 1.3127x; 1.3127x over previous
import functools
import jax
import jax.numpy as jnp
from jax.experimental import pallas as pl

_NT = ["SB", "PV", "PQ", "NB"]
_ETYPES = [("SB", "PV"), ("SB", "PQ"), ("SB", "NB"), ("PV", "PQ"),
           ("PV", "NB"), ("PQ", "NB"), ("PV", "SB"), ("PQ", "SB"),
           ("NB", "SB"), ("PQ", "PV"), ("NB", "PQ"), ("PV", "PV"),
           ("PQ", "PQ"), ("NB", "NB")]
_N = 12500
_NP = 12544
_E = 32000
_D = 128
_L = 2
_T = len(_ETYPES)
_BN = 256


def _mm_body(x_ref, w_ref, b_ref, o_ref):
    o_ref[...] = (
        jnp.dot(x_ref[...].astype(jnp.bfloat16),
                w_ref[...].astype(jnp.bfloat16),
                preferred_element_type=jnp.float32)
        + b_ref[...]
    )


def _proj(xp, w, b):
    m = w.shape[1]
    return pl.pallas_call(
        _mm_body,
        grid=(_NP // _BN,),
        in_specs=[
            pl.BlockSpec((_BN, _D), lambda i: (i, 0)),
            pl.BlockSpec((_D, m), lambda i: (0, 0)),
            pl.BlockSpec((1, m), lambda i: (0, 0)),
        ],
        out_specs=pl.BlockSpec((_BN, m), lambda i: (i, 0)),
        out_shape=jax.ShapeDtypeStruct((_NP, m), jnp.float32),
    )(xp, w, b[None, :])


def _eproj_body(ea_ref, we_ref, o_ref):
    o_ref[...] = jnp.einsum(
        "tec,tcd->ted", ea_ref[...], we_ref[...],
        preferred_element_type=jnp.float32,
        precision=jax.lax.Precision.HIGHEST)


def _eproj(ea_all, we_l):
    eb = 4000
    return pl.pallas_call(
        _eproj_body,
        grid=(_T, _E // eb),
        in_specs=[
            pl.BlockSpec((1, eb, 2), lambda t, j: (t, j, 0)),
            pl.BlockSpec((1, 2, _D), lambda t, j: (t, 0, 0)),
        ],
        out_specs=pl.BlockSpec((1, eb, _D), lambda t, j: (t, j, 0)),
        out_shape=jax.ShapeDtypeStruct((_T, _E, _D), jnp.float32),
    )(ea_all, we_l)


def _edge_stage(q, k, v, e, src, dst):
    ke = k[src] + e
    alpha = (q[dst] * ke).sum(-1) / jnp.sqrt(jnp.float32(_D))
    ex = jnp.exp(alpha)
    den = jax.ops.segment_sum(ex, dst, num_segments=_N)
    attn = ex / den[dst]
    msg = (v[src] + e) * attn[:, None]
    return jax.ops.segment_sum(msg, dst, num_segments=_N)


@jax.jit
def kernel(x_SB, x_PV, x_PQ, x_NB, ei_SB_PV, ei_SB_PQ, ei_SB_NB, ei_PV_PQ,
           ei_PV_NB, ei_PQ_NB, ei_PV_SB, ei_PQ_SB, ei_NB_SB, ei_PQ_PV,
           ei_NB_PQ, ei_PV_PV, ei_PQ_PQ, ei_NB_NB, ea_SB_PV, ea_SB_PQ,
           ea_SB_NB, ea_PV_PQ, ea_PV_NB, ea_PQ_NB, ea_PV_SB, ea_PQ_SB,
           ea_NB_SB, ea_PQ_PV, ea_NB_PQ, ea_PV_PV, ea_PQ_PQ, ea_NB_NB,
           Wq, Wk, Wv, Ws, We, bq, bk, bv, bs):
    eis = (ei_SB_PV, ei_SB_PQ, ei_SB_NB, ei_PV_PQ, ei_PV_NB, ei_PQ_NB,
           ei_PV_SB, ei_PQ_SB, ei_NB_SB, ei_PQ_PV, ei_NB_PQ, ei_PV_PV,
           ei_PQ_PQ, ei_NB_NB)
    eas = (ea_SB_PV, ea_SB_PQ, ea_SB_NB, ea_PV_PQ, ea_PV_NB, ea_PQ_NB,
           ea_PV_SB, ea_PQ_SB, ea_NB_SB, ea_PQ_PV, ea_NB_PQ, ea_PV_PV,
           ea_PQ_PQ, ea_NB_NB)
    ea_all = jnp.stack(eas)
    xd = dict(zip(_NT, (x_SB, x_PV, x_PQ, x_NB)))

    uses = {n: [] for n in _NT}
    for t, (s, d) in enumerate(_ETYPES):
        uses[d].append(("q", t))
        uses[d].append(("s", t))
        uses[s].append(("k", t))
        uses[s].append(("v", t))

    wmap = {"q": (Wq, bq), "k": (Wk, bk), "v": (Wv, bv), "s": (Ws, bs)}

    for l in range(_L):
        proj = {}
        for n in _NT:
            wcat = jnp.concatenate(
                [wmap[kind][0][l, t] for kind, t in uses[n]], axis=1)
            bcat = jnp.concatenate(
                [wmap[kind][1][l, t] for kind, t in uses[n]], axis=0)
            xp = jnp.pad(xd[n], ((0, _NP - _N), (0, 0)))
            y = _proj(xp, wcat, bcat)
            for i, (kind, t) in enumerate(uses[n]):
                proj[(kind, t)] = y[:, i * _D:(i + 1) * _D]
        eproj = _eproj(ea_all, We[l])

        acc = {n: None for n in _NT}
        for t, (s, d) in enumerate(_ETYPES):
            src = eis[t][0]
            dst = eis[t][1]
            agg = _edge_stage(proj[("q", t)], proj[("k", t)],
                              proj[("v", t)], eproj[t], src, dst)
            o = agg + proj[("s", t)][:_N]
            acc[d] = o if acc[d] is None else acc[d] + o
        xd = acc
    return tuple(xd[n] for n in _NT)


# speedup vs baseline: 1.3326x; 1.0151x over previous
import functools
import jax
import jax.numpy as jnp
from jax import lax
from jax.experimental import pallas as pl
from jax.experimental.pallas import tpu as pltpu
from jax.experimental.pallas import tpu_sc as plsc

_NT = ["SB", "PV", "PQ", "NB"]
_ETYPES = [("SB", "PV"), ("SB", "PQ"), ("SB", "NB"), ("PV", "PQ"),
           ("PV", "NB"), ("PQ", "NB"), ("PV", "SB"), ("PQ", "SB"),
           ("NB", "SB"), ("PQ", "PV"), ("NB", "PQ"), ("PV", "PV"),
           ("PQ", "PQ"), ("NB", "NB")]
_N = 12500
_NP = 12544
_E = 32000
_D = 128
_L = 2
_T = len(_ETYPES)
_BN = 256


def _mm_body(x_ref, w_ref, b_ref, o_ref):
    o_ref[...] = (
        jnp.dot(x_ref[...].astype(jnp.bfloat16),
                w_ref[...].astype(jnp.bfloat16),
                preferred_element_type=jnp.float32)
        + b_ref[...]
    )


def _proj(xp, w, b):
    m = w.shape[1]
    return pl.pallas_call(
        _mm_body,
        grid=(_NP // _BN,),
        in_specs=[
            pl.BlockSpec((_BN, _D), lambda i: (i, 0)),
            pl.BlockSpec((_D, m), lambda i: (0, 0)),
            pl.BlockSpec((1, m), lambda i: (0, 0)),
        ],
        out_specs=pl.BlockSpec((_BN, m), lambda i: (i, 0)),
        out_shape=jax.ShapeDtypeStruct((_NP, m), jnp.float32),
    )(xp, w, b[None, :])


def _eproj_body(ea_ref, we_ref, o_ref):
    o_ref[...] = jnp.einsum(
        "tec,tcd->ted", ea_ref[...], we_ref[...],
        preferred_element_type=jnp.float32,
        precision=jax.lax.Precision.HIGHEST)


def _eproj(ea_all, we_l):
    eb = 4000
    return pl.pallas_call(
        _eproj_body,
        grid=(_T, _E // eb),
        in_specs=[
            pl.BlockSpec((1, eb, 2), lambda t, j: (t, j, 0)),
            pl.BlockSpec((1, 2, _D), lambda t, j: (t, 0, 0)),
        ],
        out_specs=pl.BlockSpec((1, eb, _D), lambda t, j: (t, j, 0)),
        out_shape=jax.ShapeDtypeStruct((_T, _E, _D), jnp.float32),
    )(ea_all, we_l)


_NW = 32
_EW = _E // _NW
_CH = 125
_NCH = _EW // _CH
_NSP = 12544


def _sc_scatter_body(rows_hbm, idx_hbm, zeros_hbm, out_hbm,
                     idx_v, row_v, shared):
    c = lax.axis_index("c")
    s = lax.axis_index("s")
    wid = c * 16 + s
    @pl.when(s == 0)
    def _():
        pltpu.sync_copy(zeros_hbm, shared)
    plsc.subcore_barrier()
    for j in range(_NCH):
        pltpu.sync_copy(idx_hbm.at[wid, j], idx_v)
        pltpu.sync_copy(rows_hbm.at[wid, j], row_v)
        pltpu.sync_copy(row_v, shared.at[idx_v], add=True)
    plsc.subcore_barrier()
    pltpu.sync_copy(shared.at[pl.ds(s * 784, 784)],
                    out_hbm.at[c].at[pl.ds(s * 784, 784)])


def _sc_scatter_add(rows, idx, d):
    rows_r = rows.reshape(_NW, _NCH, _CH, d)
    idx_r = idx.astype(jnp.int32).reshape(_NW, _NCH, _CH)
    zeros = jnp.zeros((_NSP, d), jnp.float32)
    mesh = plsc.VectorSubcoreMesh(core_axis_name="c", subcore_axis_name="s")
    f = pl.kernel(
        _sc_scatter_body,
        mesh=mesh,
        out_type=jax.ShapeDtypeStruct((2, _NSP, d), jnp.float32),
        scratch_types=[
            pltpu.VMEM((_CH,), jnp.int32),
            pltpu.VMEM((_CH, d), jnp.float32),
            pltpu.VMEM_SHARED((_NSP, d), jnp.float32),
        ],
    )
    return f(rows_r, idx_r, zeros)


def _edge_stage(q, k, v, e, src, dst):
    ke = k[src] + e
    alpha = (q[dst] * ke).sum(-1) / jnp.sqrt(jnp.float32(_D))
    ex = jnp.exp(alpha)
    den2 = _sc_scatter_add(jnp.broadcast_to(ex[:, None], (_E, 16)), dst, 16)
    den = den2[0, :, 0] + den2[1, :, 0]
    attn = ex / den[dst]
    msg = (v[src] + e) * attn[:, None]
    agg2 = _sc_scatter_add(msg, dst, _D)
    return (agg2[0] + agg2[1])[:_N]


@jax.jit
def kernel(x_SB, x_PV, x_PQ, x_NB, ei_SB_PV, ei_SB_PQ, ei_SB_NB, ei_PV_PQ,
           ei_PV_NB, ei_PQ_NB, ei_PV_SB, ei_PQ_SB, ei_NB_SB, ei_PQ_PV,
           ei_NB_PQ, ei_PV_PV, ei_PQ_PQ, ei_NB_NB, ea_SB_PV, ea_SB_PQ,
           ea_SB_NB, ea_PV_PQ, ea_PV_NB, ea_PQ_NB, ea_PV_SB, ea_PQ_SB,
           ea_NB_SB, ea_PQ_PV, ea_NB_PQ, ea_PV_PV, ea_PQ_PQ, ea_NB_NB,
           Wq, Wk, Wv, Ws, We, bq, bk, bv, bs):
    eis = (ei_SB_PV, ei_SB_PQ, ei_SB_NB, ei_PV_PQ, ei_PV_NB, ei_PQ_NB,
           ei_PV_SB, ei_PQ_SB, ei_NB_SB, ei_PQ_PV, ei_NB_PQ, ei_PV_PV,
           ei_PQ_PQ, ei_NB_NB)
    eas = (ea_SB_PV, ea_SB_PQ, ea_SB_NB, ea_PV_PQ, ea_PV_NB, ea_PQ_NB,
           ea_PV_SB, ea_PQ_SB, ea_NB_SB, ea_PQ_PV, ea_NB_PQ, ea_PV_PV,
           ea_PQ_PQ, ea_NB_NB)
    ea_all = jnp.stack(eas)
    xd = dict(zip(_NT, (x_SB, x_PV, x_PQ, x_NB)))

    uses = {n: [] for n in _NT}
    for t, (s, d) in enumerate(_ETYPES):
        uses[d].append(("q", t))
        uses[d].append(("s", t))
        uses[s].append(("k", t))
        uses[s].append(("v", t))

    wmap = {"q": (Wq, bq), "k": (Wk, bk), "v": (Wv, bv), "s": (Ws, bs)}

    for l in range(_L):
        proj = {}
        for n in _NT:
            wcat = jnp.concatenate(
                [wmap[kind][0][l, t] for kind, t in uses[n]], axis=1)
            bcat = jnp.concatenate(
                [wmap[kind][1][l, t] for kind, t in uses[n]], axis=0)
            xp = jnp.pad(xd[n], ((0, _NP - _N), (0, 0)))
            y = _proj(xp, wcat, bcat)
            for i, (kind, t) in enumerate(uses[n]):
                proj[(kind, t)] = y[:, i * _D:(i + 1) * _D]
        eproj = _eproj(ea_all, We[l])

        acc = {n: None for n in _NT}
        for t, (s, d) in enumerate(_ETYPES):
            src = eis[t][0]
            dst = eis[t][1]
            agg = _edge_stage(proj[("q", t)], proj[("k", t)],
                              proj[("v", t)], eproj[t], src, dst)
            o = agg + proj[("s", t)][:_N]
            acc[d] = o if acc[d] is None else acc[d] + o
        xd = acc
    return tuple(xd[n] for n in _NT)
